# baseline (speedup 1.0000x reference)
import jax
import jax.numpy as jnp
from jax import lax
from jax.experimental import pallas as pl
from jax.experimental.pallas import tpu as pltpu
from jax.experimental.pallas import tpu_sc as plsc

VOCAB = 1000000
HID = 64
NK = HID // 16
TPAD = 128
MAXPOS = 512
B = 1024
S = 200
NW = 32
CB = 128
NBB = B // CB
CHN = S * NBB // NW


def _rsqrt_vec(x):
    i = lax.bitcast_convert_type(x, jnp.int32)
    i = jnp.full((16,), 0x5F3759DF, jnp.int32) - lax.shift_right_logical(i, 1)
    y = lax.bitcast_convert_type(i, jnp.float32)
    for _ in range(3):
        y = y * (1.5 - 0.5 * x * y * y)
    return y


def _sc_body(tokT_h, posT_ids_h, typT_ids_h, tt_h, posT_h, typT_h,
             gamma_h, beta_h, out_h,
             tids0, tids1, pids0, pids1, yids0, yids1,
             rows0, rows1, outb0, outb1, ebuf,
             posT_v, typT_v, gam_v, bet_v, stat_s, gs_s, bs_s,
             gsem0, gsem1, osem0, osem1):
    wid = lax.axis_index("s") * 2 + lax.axis_index("c")
    cid0 = wid * CHN
    tids_b = (tids0, tids1)
    pids_b = (pids0, pids1)
    yids_b = (yids0, yids1)
    rows_b = (rows0, rows1)
    out_b = (outb0, outb1)
    gsem = (gsem0, gsem1)
    osem = (osem0, osem1)

    pltpu.sync_copy(posT_h, posT_v)
    pltpu.sync_copy(typT_h, typT_v)
    pltpu.sync_copy(gamma_h, gam_v)
    pltpu.sync_copy(beta_h, bet_v)

    iota16 = lax.iota(jnp.int32, 16)
    ik = [iota16 + k * 16 for k in range(NK)]

    t0k = [plsc.load_gather(typT_v, [ik[k], jnp.zeros((16,), jnp.int32)])
           for k in range(NK)]
    t1k = [plsc.load_gather(typT_v, [ik[k], jnp.ones((16,), jnp.int32)])
           for k in range(NK)]
    for k in range(NK):
        gv = gam_v[pl.ds(k * 16, 16)]
        bv = bet_v[pl.ds(k * 16, 16)]
        for l in range(16):
            gs_s[k * 16 + l] = jnp.broadcast_to(gv[l], (16,))
            bs_s[k * 16 + l] = jnp.broadcast_to(bv[l], (16,))

    lane15 = iota16 == 15

    def fetch(c, par):
        cid = cid0 + c
        s = lax.shift_right_logical(cid, 3)
        b0 = pl.multiple_of(lax.shift_left(lax.bitwise_and(cid, 7), 7), CB)
        pltpu.sync_copy(tokT_h.at[s, pl.ds(b0, CB)], tids_b[par])
        pltpu.sync_copy(posT_ids_h.at[s, pl.ds(b0, CB)], pids_b[par])
        pltpu.sync_copy(typT_ids_h.at[s, pl.ds(b0, CB)], yids_b[par])
        pltpu.async_copy(tt_h.at[tids_b[par]], rows_b[par], gsem[par])

    def do_group(base, rows_v, pids_v, yids_v, out_v):
        for t in range(16):
            row = base + t
            tp = jnp.full((16,), row, jnp.int32)
            pid = plsc.load_gather(pids_v, [tp])
            tid = plsc.load_gather(yids_v, [tp])
            is0 = tid == 0
            e = []
            for k in range(NK):
                ty = jnp.where(is0, t0k[k], t1k[k])
                ek = (rows_v[row, pl.ds(k * 16, 16)]
                      + plsc.load_gather(posT_v, [ik[k], pid]) + ty)
                e.append(ek)
                ebuf[row, pl.ds(k * 16, 16)] = ek
            s = (e[0] + e[1]) + (e[2] + e[3])
            q = (e[0] * e[0] + e[1] * e[1]) + (e[2] * e[2] + e[3] * e[3])
            cs = plsc.cumsum(s)
            cq = plsc.cumsum(q)
            tl = jnp.full((16,), t, jnp.int32)
            plsc.store_scatter(stat_s, [tl], cs, mask=lane15)
            plsc.store_scatter(stat_s, [tl + 16], cq, mask=lane15)

        s_vec = plsc.load_gather(stat_s, [iota16])
        q_vec = plsc.load_gather(stat_s, [iota16 + 16])
        mean = s_vec * (1.0 / HID)
        var = q_vec * (1.0 / HID) - mean * mean
        r = _rsqrt_vec(var + 1e-12)
        b2 = -mean * r

        bvec = iota16 + base
        for h in range(HID):
            hl = jnp.full((16,), h, jnp.int32)
            ev = plsc.load_gather(ebuf, [bvec, hl])
            ns = ev * r + b2
            out_v[h, pl.ds(base, 16)] = ns * gs_s[h] + bs_s[h]

    def compute(par):
        rows_v = rows_b[par]
        pids_v = pids_b[par]
        yids_v = yids_b[par]
        out_v = out_b[par]

        def group_body(g, carry):
            do_group(g * 16, rows_v, pids_v, yids_v, out_v)
            return carry

        lax.fori_loop(0, CB // 16, group_body, 0)

    def owrite(c, par):
        cid = cid0 + c
        s = lax.shift_right_logical(cid, 3)
        b0 = pl.multiple_of(lax.shift_left(lax.bitwise_and(cid, 7), 7), CB)
        return out_h.at[s, :, pl.ds(b0, CB)]

    fetch(0, 0)
    fetch(1, 1)

    def pair_body(cc, carry):
        for par in range(2):
            c = cc * 2 + par
            pltpu.make_async_copy(tt_h.at[tids_b[par]], rows_b[par],
                                  gsem[par]).wait()

            @pl.when(c >= 2)
            def _():
                pltpu.make_async_copy(out_b[par], owrite(c - 2, par),
                                      osem[par]).wait()

            compute(par)
            pltpu.async_copy(out_b[par], owrite(c, par), osem[par])

            @pl.when(c + 2 < CHN)
            def _():
                fetch(c + 2, par)
        return carry

    lax.fori_loop(0, CHN // 2, pair_body, 0)

    pltpu.make_async_copy(out_b[0], owrite(CHN - 2, 0), osem[0]).wait()
    pltpu.make_async_copy(out_b[1], owrite(CHN - 1, 1), osem[1]).wait()


@jax.jit
def _run(tokT, pidsT, yidsT, tt_pad, posT, typT, gamma, beta):
    mesh = plsc.VectorSubcoreMesh(core_axis_name="c", subcore_axis_name="s")
    kern = pl.kernel(
        _sc_body,
        out_type=jax.ShapeDtypeStruct((S, HID, B), jnp.float32),
        mesh=mesh,
        compiler_params=pltpu.CompilerParams(
            needs_layout_passes=False, use_tc_tiling_on_sc=False),
        scratch_types=[
            pltpu.VMEM((CB,), jnp.int32),
            pltpu.VMEM((CB,), jnp.int32),
            pltpu.VMEM((CB,), jnp.int32),
            pltpu.VMEM((CB,), jnp.int32),
            pltpu.VMEM((CB,), jnp.int32),
            pltpu.VMEM((CB,), jnp.int32),
            pltpu.VMEM((CB, TPAD), jnp.float32),
            pltpu.VMEM((CB, TPAD), jnp.float32),
            pltpu.VMEM((HID, CB), jnp.float32),
            pltpu.VMEM((HID, CB), jnp.float32),
            pltpu.VMEM((CB, HID), jnp.float32),
            pltpu.VMEM((HID, MAXPOS), jnp.float32),
            pltpu.VMEM((HID, 2), jnp.float32),
            pltpu.VMEM((HID,), jnp.float32),
            pltpu.VMEM((HID,), jnp.float32),
            pltpu.VMEM((32,), jnp.float32),
            pltpu.VMEM((HID, 16), jnp.float32),
            pltpu.VMEM((HID, 16), jnp.float32),
            pltpu.SemaphoreType.DMA,
            pltpu.SemaphoreType.DMA,
            pltpu.SemaphoreType.DMA,
            pltpu.SemaphoreType.DMA,
        ],
    )
    return kern(tokT, pidsT, yidsT, tt_pad, posT, typT, gamma, beta)


def kernel(input_ids, position_ids, token_type_ids, token_table,
           position_table, type_table, gamma, beta):
    tt_pad = jnp.pad(token_table, ((0, 0), (0, TPAD - HID)))
    out3 = _run(input_ids.T, position_ids.T, token_type_ids.T, tt_pad,
                position_table.T, type_table.T, gamma, beta)
    return jnp.transpose(out3, (2, 0, 1))


# speedup vs baseline: 1.5731x; 1.5731x over previous
import jax
import jax.numpy as jnp
from jax import lax
from jax.experimental import pallas as pl
from jax.experimental.pallas import tpu as pltpu
from jax.experimental.pallas import tpu_sc as plsc

VOCAB = 1000000
HID = 64
NK = HID // 16
TYPES = 2
B = 1024
S = 200
NW = 32
ROWS_PW = B // NW
NG = S // 16
TAIL = S - NG * 16


def _rsqrt_vec(x):
    i = lax.bitcast_convert_type(x, jnp.int32)
    i = jnp.full((16,), 0x5F3759DF, jnp.int32) - lax.shift_right_logical(i, 1)
    y = lax.bitcast_convert_type(i, jnp.float32)
    for _ in range(3):
        y = y * (1.5 - 0.5 * x * y * y)
    return y


def _sc_body(tok_ids_h, pos_ids_h, typ_ids_h, token_table_h, pos_table_h,
             typ_table_h, gamma_h, beta_h, out_h,
             tids0, tids1, pids0, pids1, yids0, yids1,
             rows0, rows1, prows0, prows1, outb0, outb1,
             typ_v, gam_v, bet_v, stat_s,
             gsem0, gsem1, psem0, psem1, osem0, osem1):
    wid = lax.axis_index("s") * 2 + lax.axis_index("c")
    brow = wid * ROWS_PW
    tids_b = (tids0, tids1)
    pids_b = (pids0, pids1)
    yids_b = (yids0, yids1)
    rows_b = (rows0, rows1)
    prows_b = (prows0, prows1)
    out_b = (outb0, outb1)
    gsem = (gsem0, gsem1)
    psem = (psem0, psem1)
    osem = (osem0, osem1)

    pltpu.sync_copy(typ_table_h, typ_v)
    pltpu.sync_copy(gamma_h, gam_v)
    pltpu.sync_copy(beta_h, bet_v)

    t0k = [typ_v[0, pl.ds(k * 16, 16)] for k in range(NK)]
    t1k = [typ_v[1, pl.ds(k * 16, 16)] for k in range(NK)]
    gk = [gam_v[pl.ds(k * 16, 16)] for k in range(NK)]
    bk = [bet_v[pl.ds(k * 16, 16)] for k in range(NK)]

    iota16 = lax.iota(jnp.int32, 16)
    lane15 = iota16 == 15

    def fetch(c, par):
        pltpu.sync_copy(tok_ids_h.at[brow + c], tids_b[par])
        pltpu.sync_copy(pos_ids_h.at[brow + c], pids_b[par])
        pltpu.sync_copy(typ_ids_h.at[brow + c], yids_b[par])
        pltpu.async_copy(token_table_h.at[tids_b[par].at[pl.ds(0, 128)]],
                         rows_b[par].at[pl.ds(0, 128)], gsem[par])
        pltpu.async_copy(token_table_h.at[tids_b[par].at[pl.ds(128, 72)]],
                         rows_b[par].at[pl.ds(128, 72)], gsem[par])
        pltpu.async_copy(pos_table_h.at[pids_b[par].at[pl.ds(0, 128)]],
                         prows_b[par].at[pl.ds(0, 128)], psem[par])
        pltpu.async_copy(pos_table_h.at[pids_b[par].at[pl.ds(128, 72)]],
                         prows_b[par].at[pl.ds(128, 72)], psem[par])

    def wait_fetch(par):
        pltpu.make_async_copy(token_table_h.at[tids_b[par].at[pl.ds(0, 128)]],
                              rows_b[par].at[pl.ds(0, 128)], gsem[par]).wait()
        pltpu.make_async_copy(token_table_h.at[tids_b[par].at[pl.ds(128, 72)]],
                              rows_b[par].at[pl.ds(128, 72)], gsem[par]).wait()
        pltpu.make_async_copy(pos_table_h.at[pids_b[par].at[pl.ds(0, 128)]],
                              prows_b[par].at[pl.ds(0, 128)], psem[par]).wait()
        pltpu.make_async_copy(pos_table_h.at[pids_b[par].at[pl.ds(128, 72)]],
                              prows_b[par].at[pl.ds(128, 72)], psem[par]).wait()

    def do_group(base, n_tok, rows_v, prow_v, yids_v, out_v):
        for t in range(n_tok):
            row = base + t
            tp = jnp.full((16,), row, jnp.int32)
            tid = plsc.load_gather(yids_v, [tp])
            is0 = tid == 0
            e = []
            for k in range(NK):
                ty = jnp.where(is0, t0k[k], t1k[k])
                ek = (rows_v[row, pl.ds(k * 16, 16)]
                      + prow_v[row, pl.ds(k * 16, 16)] + ty)
                e.append(ek)
                out_v[row, pl.ds(k * 16, 16)] = ek
            s = (e[0] + e[1]) + (e[2] + e[3])
            q = (e[0] * e[0] + e[1] * e[1]) + (e[2] * e[2] + e[3] * e[3])
            cs = plsc.cumsum(s)
            cq = plsc.cumsum(q)
            tl = jnp.full((16,), t, jnp.int32)
            plsc.store_scatter(stat_s, [tl], cs, mask=lane15)
            plsc.store_scatter(stat_s, [tl + 16], cq, mask=lane15)

        s_vec = plsc.load_gather(stat_s, [iota16])
        q_vec = plsc.load_gather(stat_s, [iota16 + 16])
        mean = s_vec * (1.0 / HID)
        var = q_vec * (1.0 / HID) - mean * mean
        r = _rsqrt_vec(var + 1e-12)
        b2 = -mean * r
        plsc.store_scatter(stat_s, [iota16], r)
        plsc.store_scatter(stat_s, [iota16 + 16], b2)

        for t in range(n_tok):
            row = base + t
            tl = jnp.full((16,), t, jnp.int32)
            rs = plsc.load_gather(stat_s, [tl])
            bs = plsc.load_gather(stat_s, [tl + 16])
            for k in range(NK):
                ek = out_v[row, pl.ds(k * 16, 16)]
                ns = ek * rs + bs
                out_v[row, pl.ds(k * 16, 16)] = ns * gk[k] + bk[k]

    def compute(par):
        rows_v = rows_b[par]
        prow_v = prows_b[par]
        yids_v = yids_b[par]
        out_v = out_b[par]

        def group_body(g, carry):
            do_group(g * 16, 16, rows_v, prow_v, yids_v, out_v)
            return carry

        lax.fori_loop(0, NG, group_body, 0)
        do_group(NG * 16, TAIL, rows_v, prow_v, yids_v, out_v)

    fetch(0, 0)
    fetch(1, 1)

    def pair_body(cc, carry):
        for par in range(2):
            c = cc * 2 + par
            wait_fetch(par)

            @pl.when(c >= 2)
            def _():
                pltpu.make_async_copy(out_b[par], out_h.at[brow + c - 2],
                                      osem[par]).wait()

            compute(par)
            pltpu.async_copy(out_b[par], out_h.at[brow + c], osem[par])

            @pl.when(c + 2 < ROWS_PW)
            def _():
                fetch(c + 2, par)
        return carry

    lax.fori_loop(0, ROWS_PW // 2, pair_body, 0)

    pltpu.make_async_copy(out_b[0], out_h.at[brow + ROWS_PW - 2],
                          osem[0]).wait()
    pltpu.make_async_copy(out_b[1], out_h.at[brow + ROWS_PW - 1],
                          osem[1]).wait()


@jax.jit
def _run(tok_ids, pos_ids, typ_ids, token_table, pos_table, typ_table,
         gamma, beta):
    mesh = plsc.VectorSubcoreMesh(core_axis_name="c", subcore_axis_name="s")
    kern = pl.kernel(
        _sc_body,
        out_type=jax.ShapeDtypeStruct((B, S, HID), jnp.float32),
        mesh=mesh,
        compiler_params=pltpu.CompilerParams(
            needs_layout_passes=False, use_tc_tiling_on_sc=False),
        scratch_types=[
            pltpu.VMEM((S,), jnp.int32),
            pltpu.VMEM((S,), jnp.int32),
            pltpu.VMEM((S,), jnp.int32),
            pltpu.VMEM((S,), jnp.int32),
            pltpu.VMEM((S,), jnp.int32),
            pltpu.VMEM((S,), jnp.int32),
            pltpu.VMEM((S, HID), jnp.float32),
            pltpu.VMEM((S, HID), jnp.float32),
            pltpu.VMEM((S, HID), jnp.float32),
            pltpu.VMEM((S, HID), jnp.float32),
            pltpu.VMEM((S, HID), jnp.float32),
            pltpu.VMEM((S, HID), jnp.float32),
            pltpu.VMEM((TYPES, HID), jnp.float32),
            pltpu.VMEM((HID,), jnp.float32),
            pltpu.VMEM((HID,), jnp.float32),
            pltpu.VMEM((32,), jnp.float32),
            pltpu.SemaphoreType.DMA,
            pltpu.SemaphoreType.DMA,
            pltpu.SemaphoreType.DMA,
            pltpu.SemaphoreType.DMA,
            pltpu.SemaphoreType.DMA,
            pltpu.SemaphoreType.DMA,
        ],
    )
    return kern(tok_ids, pos_ids, typ_ids, token_table, pos_table, typ_table,
                gamma, beta)


def kernel(input_ids, position_ids, token_type_ids, token_table,
           position_table, type_table, gamma, beta):
    return _run(input_ids, position_ids, token_type_ids, token_table,
                position_table, type_table, gamma, beta)
